# baseline (speedup 1.0000x reference)
import jax
import jax.numpy as jnp
from jax.experimental import pallas as pl

_TOKENS = 8192
_D = 4096
_E = 64
_K = 8
_BLK = 512


def _router_kernel(x_ref, wt_ref, b_ref, sparse_ref, idx_ref):
    x = x_ref[...]
    wt = wt_ref[...]
    logits = jnp.dot(x, wt, preferred_element_type=jnp.float32) + b_ref[...]

    m = jnp.max(logits, axis=-1, keepdims=True)
    e = jnp.exp(logits - m)
    probs = e / jnp.sum(e, axis=-1, keepdims=True)

    work = probs.T
    lane = jax.lax.broadcasted_iota(jnp.int32, work.shape, 0)
    idx_rows = []
    for _ in range(_K):
        mx = jnp.max(work, axis=0, keepdims=True)
        hit = work == mx
        arg = jnp.min(jnp.where(hit, lane, _E), axis=0, keepdims=True)
        idx_rows.append(arg)
        work = jnp.where(lane == arg, -jnp.inf, work)

    sparse_ref[...] = jnp.where(jnp.isneginf(work.T), probs, 0.0)
    idx_ref[...] = jnp.concatenate(idx_rows, axis=0).T


@jax.jit
def kernel(x, W, b, training):
    del training
    wt = W.T
    b2 = b.reshape(1, _E)
    grid = (_TOKENS // _BLK,)
    sparse, idx = pl.pallas_call(
        _router_kernel,
        grid=grid,
        in_specs=[
            pl.BlockSpec((_BLK, _D), lambda i: (i, 0)),
            pl.BlockSpec((_D, _E), lambda i: (0, 0)),
            pl.BlockSpec((1, _E), lambda i: (0, 0)),
        ],
        out_specs=[
            pl.BlockSpec((_BLK, _E), lambda i: (i, 0)),
            pl.BlockSpec((_BLK, _K), lambda i: (i, 0)),
        ],
        out_shape=[
            jax.ShapeDtypeStruct((_TOKENS, _E), jnp.float32),
            jax.ShapeDtypeStruct((_TOKENS, _K), jnp.int32),
        ],
    )(x, wt, b2)
    return (sparse, idx)
